# baseline (device time: 35197 ns/iter reference)
import jax
import jax.numpy as jnp
from jax import lax
from jax.experimental import pallas as pl
from jax.experimental.pallas import tpu as pltpu

N_DEV = 16
M = 512
N = 512
CH = M // N_DEV


def kernel(A, B):
    def body(a_ref, b_ref, out_ref, partial_ref, rs_buf, chunk_ref,
             p1_send, p1_recv, p2_send, p2_recv):
        me = lax.axis_index("i")

        partial_ref[:, :] = jnp.dot(
            a_ref[:, :].astype(jnp.bfloat16),
            b_ref[:, :].astype(jnp.bfloat16),
            preferred_element_type=jnp.float32,
        )

        for j in range(N_DEV):
            jj = jnp.int32(j)

            @pl.when(me != jj)
            def _():
                rdma = pltpu.make_async_remote_copy(
                    src_ref=partial_ref.at[pl.ds(j * CH, CH), :],
                    dst_ref=rs_buf.at[me],
                    send_sem=p1_send.at[j],
                    recv_sem=p1_recv.at[me],
                    device_id=(jj,),
                    device_id_type=pl.DeviceIdType.MESH,
                )
                rdma.start()

        rs_buf[me] = partial_ref[pl.ds(me * CH, CH), :]

        for j in range(N_DEV):
            jj = jnp.int32(j)

            @pl.when(me != jj)
            def _():
                recv = pltpu.make_async_remote_copy(
                    src_ref=partial_ref.at[pl.ds(0, CH), :],
                    dst_ref=rs_buf.at[j],
                    send_sem=p1_send.at[j],
                    recv_sem=p1_recv.at[j],
                    device_id=(jj,),
                    device_id_type=pl.DeviceIdType.MESH,
                )
                recv.wait_recv()

        z = jnp.sum(rs_buf[:, :, :], axis=0)
        silu = z * (1.0 / (1.0 + jnp.exp(-z)))
        chunk_ref[:, :] = silu
        out_ref[pl.ds(me * CH, CH), :] = silu

        for j in range(N_DEV):
            jj = jnp.int32(j)

            @pl.when(me != jj)
            def _():
                rdma = pltpu.make_async_remote_copy(
                    src_ref=chunk_ref,
                    dst_ref=out_ref.at[pl.ds(me * CH, CH), :],
                    send_sem=p2_send.at[j],
                    recv_sem=p2_recv.at[me],
                    device_id=(jj,),
                    device_id_type=pl.DeviceIdType.MESH,
                )
                rdma.start()

        for j in range(N_DEV):
            jj = jnp.int32(j)

            @pl.when(me != jj)
            def _():
                recv = pltpu.make_async_remote_copy(
                    src_ref=chunk_ref,
                    dst_ref=out_ref.at[pl.ds(j * CH, CH), :],
                    send_sem=p2_send.at[j],
                    recv_sem=p2_recv.at[j],
                    device_id=(jj,),
                    device_id_type=pl.DeviceIdType.MESH,
                )
                recv.wait_recv()

                send1 = pltpu.make_async_remote_copy(
                    src_ref=partial_ref.at[pl.ds(j * CH, CH), :],
                    dst_ref=rs_buf.at[j],
                    send_sem=p1_send.at[j],
                    recv_sem=p1_recv.at[j],
                    device_id=(jj,),
                    device_id_type=pl.DeviceIdType.MESH,
                )
                send1.wait_send()

                send2 = pltpu.make_async_remote_copy(
                    src_ref=chunk_ref,
                    dst_ref=out_ref.at[pl.ds(j * CH, CH), :],
                    send_sem=p2_send.at[j],
                    recv_sem=p2_recv.at[j],
                    device_id=(jj,),
                    device_id_type=pl.DeviceIdType.MESH,
                )
                send2.wait_send()

    return pl.pallas_call(
        body,
        out_shape=jax.ShapeDtypeStruct((M, N), jnp.float32),
        in_specs=[
            pl.BlockSpec(memory_space=pltpu.VMEM),
            pl.BlockSpec(memory_space=pltpu.VMEM),
        ],
        out_specs=pl.BlockSpec(memory_space=pltpu.VMEM),
        scratch_shapes=[
            pltpu.VMEM((M, N), jnp.float32),
            pltpu.VMEM((N_DEV, CH, N), jnp.float32),
            pltpu.VMEM((CH, N), jnp.float32),
            pltpu.SemaphoreType.DMA((N_DEV,)),
            pltpu.SemaphoreType.DMA((N_DEV,)),
            pltpu.SemaphoreType.DMA((N_DEV,)),
            pltpu.SemaphoreType.DMA((N_DEV,)),
        ],
    )(A, B)


# device time: 27086 ns/iter; 1.2995x vs baseline; 1.2995x over previous
import jax
import jax.numpy as jnp
from jax import lax
from jax.experimental import pallas as pl
from jax.experimental.pallas import tpu as pltpu

N_DEV = 16
M = 512
N = 512
CH = M // N_DEV


def kernel(A, B):
    def body(a_ref, b_ref, out_ref, partial_ref, rs_buf, chunk_ref, ag_buf,
             p1_send, p1_recv, p2_send, p2_recv):
        me = lax.axis_index("i")

        partial_ref[:, :] = jnp.dot(
            a_ref[:, :].astype(jnp.bfloat16),
            b_ref[:, :].astype(jnp.bfloat16),
            preferred_element_type=jnp.float32,
        ).astype(jnp.bfloat16)

        for j in range(N_DEV):
            jj = jnp.int32(j)

            @pl.when(me != jj)
            def _():
                rdma = pltpu.make_async_remote_copy(
                    src_ref=partial_ref.at[pl.ds(j * CH, CH), :],
                    dst_ref=rs_buf.at[me],
                    send_sem=p1_send.at[j],
                    recv_sem=p1_recv.at[me],
                    device_id=(jj,),
                    device_id_type=pl.DeviceIdType.MESH,
                )
                rdma.start()

        rs_buf[me] = partial_ref[pl.ds(me * CH, CH), :]

        for j in range(N_DEV):
            jj = jnp.int32(j)

            @pl.when(me != jj)
            def _():
                recv = pltpu.make_async_remote_copy(
                    src_ref=partial_ref.at[pl.ds(0, CH), :],
                    dst_ref=rs_buf.at[j],
                    send_sem=p1_send.at[j],
                    recv_sem=p1_recv.at[j],
                    device_id=(jj,),
                    device_id_type=pl.DeviceIdType.MESH,
                )
                recv.wait_recv()

        z = jnp.sum(rs_buf[:, :, :].astype(jnp.float32), axis=0)
        silu = z * (1.0 / (1.0 + jnp.exp(-z)))
        chunk_ref[:, :] = silu.astype(jnp.bfloat16)
        ag_buf[pl.ds(me * CH, CH), :] = chunk_ref[:, :]

        for j in range(N_DEV):
            jj = jnp.int32(j)

            @pl.when(me != jj)
            def _():
                rdma = pltpu.make_async_remote_copy(
                    src_ref=chunk_ref,
                    dst_ref=ag_buf.at[pl.ds(me * CH, CH), :],
                    send_sem=p2_send.at[j],
                    recv_sem=p2_recv.at[me],
                    device_id=(jj,),
                    device_id_type=pl.DeviceIdType.MESH,
                )
                rdma.start()

        for j in range(N_DEV):
            jj = jnp.int32(j)

            @pl.when(me != jj)
            def _():
                recv = pltpu.make_async_remote_copy(
                    src_ref=chunk_ref,
                    dst_ref=ag_buf.at[pl.ds(j * CH, CH), :],
                    send_sem=p2_send.at[j],
                    recv_sem=p2_recv.at[j],
                    device_id=(jj,),
                    device_id_type=pl.DeviceIdType.MESH,
                )
                recv.wait_recv()

                send1 = pltpu.make_async_remote_copy(
                    src_ref=partial_ref.at[pl.ds(j * CH, CH), :],
                    dst_ref=rs_buf.at[j],
                    send_sem=p1_send.at[j],
                    recv_sem=p1_recv.at[j],
                    device_id=(jj,),
                    device_id_type=pl.DeviceIdType.MESH,
                )
                send1.wait_send()

                send2 = pltpu.make_async_remote_copy(
                    src_ref=chunk_ref,
                    dst_ref=ag_buf.at[pl.ds(j * CH, CH), :],
                    send_sem=p2_send.at[j],
                    recv_sem=p2_recv.at[j],
                    device_id=(jj,),
                    device_id_type=pl.DeviceIdType.MESH,
                )
                send2.wait_send()

        out_ref[:, :] = ag_buf[:, :].astype(jnp.float32)

    return pl.pallas_call(
        body,
        out_shape=jax.ShapeDtypeStruct((M, N), jnp.float32),
        in_specs=[
            pl.BlockSpec(memory_space=pltpu.VMEM),
            pl.BlockSpec(memory_space=pltpu.VMEM),
        ],
        out_specs=pl.BlockSpec(memory_space=pltpu.VMEM),
        scratch_shapes=[
            pltpu.VMEM((M, N), jnp.bfloat16),
            pltpu.VMEM((N_DEV, CH, N), jnp.bfloat16),
            pltpu.VMEM((CH, N), jnp.bfloat16),
            pltpu.VMEM((M, N), jnp.bfloat16),
            pltpu.SemaphoreType.DMA((N_DEV,)),
            pltpu.SemaphoreType.DMA((N_DEV,)),
            pltpu.SemaphoreType.DMA((N_DEV,)),
            pltpu.SemaphoreType.DMA((N_DEV,)),
        ],
    )(A, B)
